# baseline (speedup 1.0000x reference)
import functools

import jax
import jax.numpy as jnp
from jax import lax
from jax.experimental import pallas as pl
from jax.experimental.pallas import tpu as pltpu
from jax.experimental.pallas import tpu_sc as plsc

SLOTS = 8
LAG = 4
CHUNK = 512


def _cu_seqlens_on_sc(past_cu, cur_cu):
    n = past_cu.shape[0]
    mesh = plsc.VectorSubcoreMesh(core_axis_name="c", subcore_axis_name="s")

    @functools.partial(
        pl.kernel,
        mesh=mesh,
        out_type=jax.ShapeDtypeStruct(past_cu.shape, past_cu.dtype),
        scratch_types=[
            pltpu.VMEM((16,), jnp.int32),
            pltpu.VMEM((16,), jnp.int32),
            pltpu.VMEM((16,), jnp.int32),
            pltpu.SemaphoreType.DMA,
            pltpu.SemaphoreType.DMA,
        ],
    )
    def add(pcu, ccu, ncu, a_v, b_v, o_v, s0, s1):
        wid = lax.axis_index("s") * 2 + lax.axis_index("c")

        @pl.when(wid == 0)
        def _():
            cp_a = pltpu.make_async_copy(pcu, a_v.at[pl.ds(0, n)], s0)
            cp_b = pltpu.make_async_copy(ccu, b_v.at[pl.ds(0, n)], s1)
            cp_a.start()
            cp_b.start()
            cp_a.wait()
            cp_b.wait()
            o_v[...] = a_v[...] + b_v[...]
            pltpu.sync_copy(o_v.at[pl.ds(0, n)], ncu)

    return add(past_cu, cur_cu)


def kernel(past_k, past_v, past_cu_seqlens, cur_k, cur_v, cur_cu_seqlens):
    nb = past_cu_seqlens.shape[0] - 1
    past_len = past_k.shape[0] // nb
    cur_len = cur_k.shape[0] // nb
    new_len = past_len + cur_len
    tail = past_k.shape[1:]
    total_new = nb * new_len
    per_seq = past_len // CHUNK

    def body(pk, pv, ck, cv, nk, nv, bufs, in_sems, out_sems):
        jobs = []
        for src, cur, dst in ((pk, ck, nk), (pv, cv, nv)):
            for b in range(nb):
                for c in range(per_seq):
                    jobs.append((src, b * past_len + c * CHUNK,
                                 dst, b * new_len + c * CHUNK, CHUNK))
                jobs.append((cur, b * cur_len,
                             dst, b * new_len + past_len, cur_len))

        def read(j, s):
            src, so, _, _, n = jobs[j]
            return pltpu.make_async_copy(
                src.at[pl.ds(so, n)], bufs.at[s, pl.ds(0, n)], in_sems.at[s])

        def write(j, s):
            _, _, dst, do, n = jobs[j]
            return pltpu.make_async_copy(
                bufs.at[s, pl.ds(0, n)], dst.at[pl.ds(do, n)], out_sems.at[s])

        nj = len(jobs)
        for j in range(nj):
            s = j % SLOTS
            if j >= SLOTS:
                write(j - SLOTS, s).wait()
            read(j, s).start()
            if j >= LAG:
                w = j - LAG
                read(w, w % SLOTS).wait()
                write(w, w % SLOTS).start()
        for w in range(nj - LAG, nj):
            read(w, w % SLOTS).wait()
            write(w, w % SLOTS).start()
        for w in range(nj - SLOTS, nj):
            write(w, w % SLOTS).wait()

    any_spec = pl.BlockSpec(memory_space=pl.ANY)

    new_cu = _cu_seqlens_on_sc(past_cu_seqlens, cur_cu_seqlens)

    new_k, new_v = pl.pallas_call(
        body,
        in_specs=[any_spec] * 4,
        out_specs=[any_spec, any_spec],
        out_shape=[
            jax.ShapeDtypeStruct((total_new,) + tail, past_k.dtype),
            jax.ShapeDtypeStruct((total_new,) + tail, past_v.dtype),
        ],
        scratch_shapes=[
            pltpu.VMEM((SLOTS, CHUNK) + tail, past_k.dtype),
            pltpu.SemaphoreType.DMA((SLOTS,)),
            pltpu.SemaphoreType.DMA((SLOTS,)),
        ],
    )(past_k, past_v, cur_k, cur_v)

    return new_k, new_v, new_cu


# speedup vs baseline: 1.0162x; 1.0162x over previous
import functools

import jax
import jax.numpy as jnp
from jax import lax
from jax.experimental import pallas as pl
from jax.experimental.pallas import tpu as pltpu
from jax.experimental.pallas import tpu_sc as plsc

SLOTS = 8
LAG = 4
CHUNK = 512


def _cu_seqlens_on_sc(past_cu, cur_cu):
    n = past_cu.shape[0]
    mesh = plsc.VectorSubcoreMesh(core_axis_name="c", subcore_axis_name="s",
                                  num_cores=1)

    @functools.partial(
        pl.kernel,
        mesh=mesh,
        out_type=jax.ShapeDtypeStruct(past_cu.shape, past_cu.dtype),
        scratch_types=[
            pltpu.VMEM((16,), jnp.int32),
            pltpu.VMEM((16,), jnp.int32),
            pltpu.VMEM((16,), jnp.int32),
            pltpu.SemaphoreType.DMA,
            pltpu.SemaphoreType.DMA,
        ],
    )
    def add(pcu, ccu, ncu, a_v, b_v, o_v, s0, s1):
        wid = lax.axis_index("s") * 2 + lax.axis_index("c")

        @pl.when(wid == 0)
        def _():
            cp_a = pltpu.make_async_copy(pcu, a_v.at[pl.ds(0, n)], s0)
            cp_b = pltpu.make_async_copy(ccu, b_v.at[pl.ds(0, n)], s1)
            cp_a.start()
            cp_b.start()
            cp_a.wait()
            cp_b.wait()
            o_v[...] = a_v[...] + b_v[...]
            pltpu.sync_copy(o_v.at[pl.ds(0, n)], ncu)

    return add(past_cu, cur_cu)


def kernel(past_k, past_v, past_cu_seqlens, cur_k, cur_v, cur_cu_seqlens):
    nb = past_cu_seqlens.shape[0] - 1
    past_len = past_k.shape[0] // nb
    cur_len = cur_k.shape[0] // nb
    new_len = past_len + cur_len
    tail = past_k.shape[1:]
    total_new = nb * new_len
    per_seq = past_len // CHUNK

    def body(pk, pv, ck, cv, nk, nv, bufs, in_sems, out_sems):
        jobs = []
        for src, cur, dst in ((pk, ck, nk), (pv, cv, nv)):
            for b in range(nb):
                for c in range(per_seq):
                    jobs.append((src, b * past_len + c * CHUNK,
                                 dst, b * new_len + c * CHUNK, CHUNK))
                jobs.append((cur, b * cur_len,
                             dst, b * new_len + past_len, cur_len))

        def read(j, s):
            src, so, _, _, n = jobs[j]
            return pltpu.make_async_copy(
                src.at[pl.ds(so, n)], bufs.at[s, pl.ds(0, n)], in_sems.at[s])

        def write(j, s):
            _, _, dst, do, n = jobs[j]
            return pltpu.make_async_copy(
                bufs.at[s, pl.ds(0, n)], dst.at[pl.ds(do, n)], out_sems.at[s])

        nj = len(jobs)
        for j in range(nj):
            s = j % SLOTS
            if j >= SLOTS:
                write(j - SLOTS, s).wait()
            read(j, s).start()
            if j >= LAG:
                w = j - LAG
                read(w, w % SLOTS).wait()
                write(w, w % SLOTS).start()
        for w in range(nj - LAG, nj):
            read(w, w % SLOTS).wait()
            write(w, w % SLOTS).start()
        for w in range(nj - SLOTS, nj):
            write(w, w % SLOTS).wait()

    any_spec = pl.BlockSpec(memory_space=pl.ANY)

    new_cu = _cu_seqlens_on_sc(past_cu_seqlens, cur_cu_seqlens)

    new_k, new_v = pl.pallas_call(
        body,
        in_specs=[any_spec] * 4,
        out_specs=[any_spec, any_spec],
        out_shape=[
            jax.ShapeDtypeStruct((total_new,) + tail, past_k.dtype),
            jax.ShapeDtypeStruct((total_new,) + tail, past_v.dtype),
        ],
        scratch_shapes=[
            pltpu.VMEM((SLOTS, CHUNK) + tail, past_k.dtype),
            pltpu.SemaphoreType.DMA((SLOTS,)),
            pltpu.SemaphoreType.DMA((SLOTS,)),
        ],
    )(past_k, past_v, cur_k, cur_v)

    return new_k, new_v, new_cu


# speedup vs baseline: 1.0164x; 1.0002x over previous
import functools

import jax
import jax.numpy as jnp
from jax import lax
from jax.experimental import pallas as pl
from jax.experimental.pallas import tpu as pltpu
from jax.experimental.pallas import tpu_sc as plsc

SLOTS = 8
LAG = 4
CHUNK = 512


def _cu_seqlens_on_sc(past_cu, cur_cu):
    n = past_cu.shape[0]
    mesh = plsc.ScalarSubcoreMesh(axis_name="c", num_cores=1)

    @functools.partial(
        pl.kernel,
        mesh=mesh,
        out_type=jax.ShapeDtypeStruct(past_cu.shape, past_cu.dtype),
        scratch_types=[
            pltpu.SMEM((n,), jnp.int32),
            pltpu.SMEM((n,), jnp.int32),
            pltpu.SMEM((n,), jnp.int32),
            pltpu.SemaphoreType.DMA,
            pltpu.SemaphoreType.DMA,
        ],
    )
    def add(pcu, ccu, ncu, a_s, b_s, o_s, s0, s1):
        cp_a = pltpu.make_async_copy(pcu, a_s, s0)
        cp_b = pltpu.make_async_copy(ccu, b_s, s1)
        cp_a.start()
        cp_b.start()
        cp_a.wait()
        cp_b.wait()
        for i in range(n):
            o_s[i] = a_s[i] + b_s[i]
        pltpu.sync_copy(o_s, ncu)

    return add(past_cu, cur_cu)


def kernel(past_k, past_v, past_cu_seqlens, cur_k, cur_v, cur_cu_seqlens):
    nb = past_cu_seqlens.shape[0] - 1
    past_len = past_k.shape[0] // nb
    cur_len = cur_k.shape[0] // nb
    new_len = past_len + cur_len
    tail = past_k.shape[1:]
    total_new = nb * new_len
    per_seq = past_len // CHUNK

    def body(pk, pv, ck, cv, nk, nv, bufs, in_sems, out_sems):
        jobs = []
        for src, cur, dst in ((pk, ck, nk), (pv, cv, nv)):
            for b in range(nb):
                for c in range(per_seq):
                    jobs.append((src, b * past_len + c * CHUNK,
                                 dst, b * new_len + c * CHUNK, CHUNK))
                jobs.append((cur, b * cur_len,
                             dst, b * new_len + past_len, cur_len))

        def read(j, s):
            src, so, _, _, n = jobs[j]
            return pltpu.make_async_copy(
                src.at[pl.ds(so, n)], bufs.at[s, pl.ds(0, n)], in_sems.at[s])

        def write(j, s):
            _, _, dst, do, n = jobs[j]
            return pltpu.make_async_copy(
                bufs.at[s, pl.ds(0, n)], dst.at[pl.ds(do, n)], out_sems.at[s])

        nj = len(jobs)
        for j in range(nj):
            s = j % SLOTS
            if j >= SLOTS:
                write(j - SLOTS, s).wait()
            read(j, s).start()
            if j >= LAG:
                w = j - LAG
                read(w, w % SLOTS).wait()
                write(w, w % SLOTS).start()
        for w in range(nj - LAG, nj):
            read(w, w % SLOTS).wait()
            write(w, w % SLOTS).start()
        for w in range(nj - SLOTS, nj):
            write(w, w % SLOTS).wait()

    any_spec = pl.BlockSpec(memory_space=pl.ANY)

    new_cu = _cu_seqlens_on_sc(past_cu_seqlens, cur_cu_seqlens)

    new_k, new_v = pl.pallas_call(
        body,
        in_specs=[any_spec] * 4,
        out_specs=[any_spec, any_spec],
        out_shape=[
            jax.ShapeDtypeStruct((total_new,) + tail, past_k.dtype),
            jax.ShapeDtypeStruct((total_new,) + tail, past_v.dtype),
        ],
        scratch_shapes=[
            pltpu.VMEM((SLOTS, CHUNK) + tail, past_k.dtype),
            pltpu.SemaphoreType.DMA((SLOTS,)),
            pltpu.SemaphoreType.DMA((SLOTS,)),
        ],
    )(past_k, past_v, cur_k, cur_v)

    return new_k, new_v, new_cu


# speedup vs baseline: 1.0170x; 1.0005x over previous
import functools

import jax
import jax.numpy as jnp
from jax.experimental import pallas as pl
from jax.experimental.pallas import tpu as pltpu
from jax.experimental.pallas import tpu_sc as plsc

SLOTS = 8
LAG = 4
CHUNK = 512


def _cu_seqlens_on_sc(past_cu, cur_cu):
    n = past_cu.shape[0]
    mesh = plsc.ScalarSubcoreMesh(axis_name="c", num_cores=1)

    @functools.partial(
        pl.kernel,
        mesh=mesh,
        out_type=jax.ShapeDtypeStruct(past_cu.shape, past_cu.dtype),
        scratch_types=[
            pltpu.SMEM((n,), jnp.int32),
            pltpu.SMEM((n,), jnp.int32),
            pltpu.SMEM((n,), jnp.int32),
            pltpu.SemaphoreType.DMA,
            pltpu.SemaphoreType.DMA,
        ],
    )
    def add(pcu, ccu, ncu, a_s, b_s, o_s, s0, s1):
        cp_a = pltpu.make_async_copy(pcu, a_s, s0)
        cp_b = pltpu.make_async_copy(ccu, b_s, s1)
        cp_a.start()
        cp_b.start()
        cp_a.wait()
        cp_b.wait()
        for i in range(n):
            o_s[i] = a_s[i] + b_s[i]
        pltpu.sync_copy(o_s, ncu)

    return add(past_cu, cur_cu)


def kernel(past_k, past_v, past_cu_seqlens, cur_k, cur_v, cur_cu_seqlens):
    nb = past_cu_seqlens.shape[0] - 1
    past_len = past_k.shape[0] // nb
    cur_len = cur_k.shape[0] // nb
    new_len = past_len + cur_len
    tail = past_k.shape[1:]
    total_new = nb * new_len
    per_seq = past_len // CHUNK

    def body(pk, pv, ck, cv, nk, nv, bufs, in_sems, out_sems):
        jobs = []
        for src, cur, dst in ((pk, ck, nk), (pv, cv, nv)):
            for b in range(nb):
                for c in range(per_seq):
                    jobs.append((src, b * past_len + c * CHUNK,
                                 dst, b * new_len + c * CHUNK, CHUNK))
                jobs.append((cur, b * cur_len,
                             dst, b * new_len + past_len, cur_len))

        def read(j, s):
            src, so, _, _, n = jobs[j]
            return pltpu.make_async_copy(
                src.at[pl.ds(so, n)], bufs.at[s, pl.ds(0, n)], in_sems.at[s])

        def write(j, s):
            _, _, dst, do, n = jobs[j]
            return pltpu.make_async_copy(
                bufs.at[s, pl.ds(0, n)], dst.at[pl.ds(do, n)], out_sems.at[s])

        nj = len(jobs)
        for j in range(nj):
            s = j % SLOTS
            if j >= SLOTS:
                write(j - SLOTS, s).wait()
            read(j, s).start()
            if j >= LAG:
                w = j - LAG
                read(w, w % SLOTS).wait()
                write(w, w % SLOTS).start()
        for w in range(nj - LAG, nj):
            read(w, w % SLOTS).wait()
            write(w, w % SLOTS).start()
        for w in range(nj - SLOTS, nj):
            write(w, w % SLOTS).wait()

    any_spec = pl.BlockSpec(memory_space=pl.ANY)

    new_cu = _cu_seqlens_on_sc(past_cu_seqlens, cur_cu_seqlens)

    new_k, new_v = pl.pallas_call(
        body,
        in_specs=[any_spec] * 4,
        out_specs=[any_spec, any_spec],
        out_shape=[
            jax.ShapeDtypeStruct((total_new,) + tail, past_k.dtype),
            jax.ShapeDtypeStruct((total_new,) + tail, past_v.dtype),
        ],
        scratch_shapes=[
            pltpu.VMEM((SLOTS, CHUNK) + tail, past_k.dtype),
            pltpu.SemaphoreType.DMA((SLOTS,)),
            pltpu.SemaphoreType.DMA((SLOTS,)),
        ],
    )(past_k, past_v, cur_k, cur_v)

    return new_k, new_v, new_cu
